# baseline (speedup 1.0000x reference)
import functools

import jax
import jax.numpy as jnp
from jax import lax
from jax.experimental import pallas as pl
from jax.experimental.pallas import tpu as pltpu
from jax.experimental.pallas import tpu_sc as plsc

B, S, H, D = 4, 8192, 2048, 16
NC, NS, L = 2, 16, 16
NW = NC * NS
ROWS = B * S
RPW = ROWS // NW
RCH = 16
NCHUNK = RPW // RCH

@functools.cache
def _make_pool_sc():
    mesh = plsc.VectorSubcoreMesh(core_axis_name="c", subcore_axis_name="s")
    return functools.partial(
        pl.kernel,
        mesh=mesh,
        out_type=jax.ShapeDtypeStruct((NW, H), jnp.float32),
        scratch_types=[
            pltpu.VMEM((RCH, H), jnp.float32),
            pltpu.VMEM((RCH, H), jnp.float32),
            pltpu.VMEM((H,), jnp.float32),
            pltpu.SemaphoreType.DMA,
            pltpu.SemaphoreType.DMA,
        ],
    )(_pool_sc_body)


def _pool_sc_body(h_hbm, out_hbm, buf0, buf1, acc, sem0, sem1):
    cid = lax.axis_index("c")
    sid = lax.axis_index("s")
    wid = sid * NC + cid
    base = wid * RPW

    def _start(buf, sem, chunk):
        pltpu.make_async_copy(
            h_hbm.at[pl.ds(base + chunk * RCH, RCH)], buf, sem
        ).start()

    def _wait(buf, sem):
        pltpu.make_async_copy(h_hbm.at[pl.ds(base, RCH)], buf, sem).wait()

    def _zero(g, _):
        acc[pl.ds(g * L, L)] = jnp.zeros((L,), jnp.float32)
        return 0

    lax.fori_loop(0, H // L, _zero, 0)

    def _accum(buf):
        GU = 8

        def _one(g):
            col = pl.ds(g * L, L)
            x = [buf[r, col] for r in range(RCH)]
            while len(x) > 1:
                x = [x[i] + x[i + 1] for i in range(0, len(x), 2)]
            acc[col] = acc[col] + x[0]

        def _g(g, _):
            for u in range(GU):
                _one(g * GU + u)
            return 0

        lax.fori_loop(0, H // (L * GU), _g, 0)

    _start(buf0, sem0, 0)

    def _body(i, _):
        c0 = i * 2
        _start(buf1, sem1, c0 + 1)
        _wait(buf0, sem0)
        _accum(buf0)

        @pl.when(i < NCHUNK // 2 - 1)
        def _():
            _start(buf0, sem0, c0 + 2)

        _wait(buf1, sem1)
        _accum(buf1)
        return 0

    lax.fori_loop(0, NCHUNK // 2, _body, 0)
    pltpu.sync_copy(acc, out_hbm.at[wid])


def _gate_body(p_ref, w_ref, b_ref, loss_ref, idx_ref):
    pooled = jnp.sum(p_ref[...], axis=1) * (1.0 / S)
    logits = (
        jnp.dot(pooled, w_ref[...], preferred_element_type=jnp.float32)
        + b_ref[...]
    )
    m = jnp.max(logits, axis=1, keepdims=True)
    e = jnp.exp(logits - m)
    probs = e / jnp.sum(e, axis=1, keepdims=True)
    entropy = -jnp.sum(probs * jnp.log(probs + 1e-10)) * (1.0 / B)
    loss_ref[...] = jnp.reshape(-0.01 * entropy, (1, 1))
    row0 = probs[0:1, :]
    iota = lax.broadcasted_iota(jnp.int32, (1, D), 1)
    mx = jnp.max(row0)
    idx_ref[...] = jnp.reshape(jnp.min(jnp.where(row0 == mx, iota, D)), (1, 1))


def kernel(hidden_states, gates_W, gates_b, current_depth):
    h2 = hidden_states.reshape(ROWS, H)
    partials = _make_pool_sc()(h2)
    w_d = lax.dynamic_index_in_dim(gates_W, current_depth, 0, keepdims=False)
    b_d = lax.dynamic_index_in_dim(gates_b, current_depth, 0, keepdims=True)
    loss, idx = pl.pallas_call(
        _gate_body,
        out_shape=(
            jax.ShapeDtypeStruct((1, 1), jnp.float32),
            jax.ShapeDtypeStruct((1, 1), jnp.int32),
        ),
    )(partials.reshape(B, NW // B, H), w_d, b_d)
    return (loss[0, 0], idx[0, 0])


# speedup vs baseline: 1.6399x; 1.6399x over previous
import functools

import jax
import jax.numpy as jnp
from jax import lax
from jax.experimental import pallas as pl
from jax.experimental.pallas import tpu as pltpu
from jax.experimental.pallas import tpu_sc as plsc

B, S, H, D = 4, 8192, 2048, 16
NC, NS, L = 2, 16, 16
NW = NC * NS
WPB = NW // B

S_SC = 2560
SPW = S_SC // WPB
RCH = 16
NCH = SPW // RCH

CS = 512
NT = (S - S_SC) // CS


@functools.cache
def _make_pool_sc():
    mesh = plsc.VectorSubcoreMesh(core_axis_name="c", subcore_axis_name="s")
    return functools.partial(
        pl.kernel,
        mesh=mesh,
        out_type=jax.ShapeDtypeStruct((NW, H), jnp.float32),
        scratch_types=[
            pltpu.VMEM((RCH, H), jnp.float32),
            pltpu.VMEM((RCH, H), jnp.float32),
            pltpu.VMEM((H,), jnp.float32),
            pltpu.SemaphoreType.DMA,
            pltpu.SemaphoreType.DMA,
        ],
    )(_pool_sc_body)


def _pool_sc_body(h_hbm, out_hbm, buf0, buf1, acc, sem0, sem1):
    cid = lax.axis_index("c")
    sid = lax.axis_index("s")
    wid = sid * NC + cid
    b = wid // WPB
    base = b * S + (wid % WPB) * SPW

    def _start(buf, sem, chunk):
        pltpu.make_async_copy(
            h_hbm.at[pl.ds(base + chunk * RCH, RCH)], buf, sem
        ).start()

    def _wait(buf, sem):
        pltpu.make_async_copy(h_hbm.at[pl.ds(base, RCH)], buf, sem).wait()

    def _zero(g, _):
        acc[pl.ds(g * L, L)] = jnp.zeros((L,), jnp.float32)
        return 0

    lax.fori_loop(0, H // L, _zero, 0)

    def _accum(buf):
        GU = 2

        def _one(g):
            col = pl.ds(g * L, L)
            x = [buf[r, col] for r in range(RCH)]
            while len(x) > 1:
                x = [x[i] + x[i + 1] for i in range(0, len(x), 2)]
            acc[col] = acc[col] + x[0]

        def _g(g, _):
            for u in range(GU):
                _one(g * GU + u)
            return 0

        lax.fori_loop(0, H // (L * GU), _g, 0)

    _start(buf0, sem0, 0)

    def _body(i, _):
        c0 = i * 2
        _start(buf1, sem1, c0 + 1)
        _wait(buf0, sem0)
        _accum(buf0)

        @pl.when(i < NCH // 2 - 1)
        def _():
            _start(buf0, sem0, c0 + 2)

        _wait(buf1, sem1)
        _accum(buf1)
        return 0

    lax.fori_loop(0, NCH // 2, _body, 0)
    pltpu.sync_copy(acc, out_hbm.at[wid])


def _tc_pool_body(h_ref, out_ref):
    @pl.when(pl.program_id(0) == 0)
    def _():
        out_ref[...] = jnp.zeros_like(out_ref)

    out_ref[...] += jnp.sum(h_ref[...], axis=1)


def _tc_pool(h3):
    return pl.pallas_call(
        _tc_pool_body,
        grid=(NT,),
        in_specs=[
            pl.BlockSpec((B, CS, H), lambda i: (0, S_SC // CS + i, 0)),
        ],
        out_specs=pl.BlockSpec((B, H), lambda i: (0, 0)),
        out_shape=jax.ShapeDtypeStruct((B, H), jnp.float32),
    )(h3)


def _gate_body(p_ref, t_ref, w_ref, b_ref, loss_ref, idx_ref):
    pooled = (jnp.sum(p_ref[...], axis=1) + t_ref[...]) * (1.0 / S)
    logits = (
        jnp.dot(pooled, w_ref[...], preferred_element_type=jnp.float32)
        + b_ref[...]
    )
    m = jnp.max(logits, axis=1, keepdims=True)
    e = jnp.exp(logits - m)
    probs = e / jnp.sum(e, axis=1, keepdims=True)
    entropy = -jnp.sum(probs * jnp.log(probs + 1e-10)) * (1.0 / B)
    loss_ref[...] = jnp.reshape(-0.01 * entropy, (1, 1))
    row0 = probs[0:1, :]
    iota = lax.broadcasted_iota(jnp.int32, (1, D), 1)
    mx = jnp.max(row0)
    idx_ref[...] = jnp.reshape(jnp.min(jnp.where(row0 == mx, iota, D)), (1, 1))


def kernel(hidden_states, gates_W, gates_b, current_depth):
    h2 = hidden_states.reshape(B * S, H)
    sc_partials = _make_pool_sc()(h2)
    tc_sums = _tc_pool(hidden_states)
    w_d = lax.dynamic_index_in_dim(gates_W, current_depth, 0, keepdims=False)
    b_d = lax.dynamic_index_in_dim(gates_b, current_depth, 0, keepdims=True)
    loss, idx = pl.pallas_call(
        _gate_body,
        out_shape=(
            jax.ShapeDtypeStruct((1, 1), jnp.float32),
            jax.ShapeDtypeStruct((1, 1), jnp.int32),
        ),
    )(sc_partials.reshape(B, WPB, H), tc_sums, w_d, b_d)
    return (loss[0, 0], idx[0, 0])


# speedup vs baseline: 1.6738x; 1.0206x over previous
import functools

import jax
import jax.numpy as jnp
from jax import lax
from jax.experimental import pallas as pl
from jax.experimental.pallas import tpu as pltpu
from jax.experimental.pallas import tpu_sc as plsc

B, S, H, D = 4, 8192, 2048, 16
NC, NS, L = 2, 16, 16
NW = NC * NS
WPB = NW // B

S_SC = 1024
SPW = S_SC // WPB
RCH = 16
NCH = SPW // RCH

CS = 1024
NT = (S - S_SC) // CS


@functools.cache
def _make_pool_sc():
    mesh = plsc.VectorSubcoreMesh(core_axis_name="c", subcore_axis_name="s")
    return functools.partial(
        pl.kernel,
        mesh=mesh,
        out_type=jax.ShapeDtypeStruct((NW, H), jnp.float32),
        scratch_types=[
            pltpu.VMEM((RCH, H), jnp.float32),
            pltpu.VMEM((RCH, H), jnp.float32),
            pltpu.VMEM((H,), jnp.float32),
            pltpu.SemaphoreType.DMA,
            pltpu.SemaphoreType.DMA,
        ],
    )(_pool_sc_body)


def _pool_sc_body(h_hbm, out_hbm, buf0, buf1, acc, sem0, sem1):
    cid = lax.axis_index("c")
    sid = lax.axis_index("s")
    wid = sid * NC + cid
    b = wid // WPB
    base = b * S + (wid % WPB) * SPW

    def _start(buf, sem, chunk):
        pltpu.make_async_copy(
            h_hbm.at[pl.ds(base + chunk * RCH, RCH)], buf, sem
        ).start()

    def _wait(buf, sem):
        pltpu.make_async_copy(h_hbm.at[pl.ds(base, RCH)], buf, sem).wait()

    def _zero(g, _):
        acc[pl.ds(g * L, L)] = jnp.zeros((L,), jnp.float32)
        return 0

    lax.fori_loop(0, H // L, _zero, 0)

    def _accum(buf):
        GU = 2

        def _one(g):
            col = pl.ds(g * L, L)
            x = [buf[r, col] for r in range(RCH)]
            while len(x) > 1:
                x = [x[i] + x[i + 1] for i in range(0, len(x), 2)]
            acc[col] = acc[col] + x[0]

        def _g(g, _):
            for u in range(GU):
                _one(g * GU + u)
            return 0

        lax.fori_loop(0, H // (L * GU), _g, 0)

    _start(buf0, sem0, 0)

    def _body(i, _):
        c0 = i * 2
        _start(buf1, sem1, c0 + 1)
        _wait(buf0, sem0)
        _accum(buf0)

        @pl.when(i < NCH // 2 - 1)
        def _():
            _start(buf0, sem0, c0 + 2)

        _wait(buf1, sem1)
        _accum(buf1)
        return 0

    lax.fori_loop(0, NCH // 2, _body, 0)
    pltpu.sync_copy(acc, out_hbm.at[wid])


def _tc_pool_body(h_ref, out_ref):
    @pl.when(pl.program_id(0) == 0)
    def _():
        out_ref[...] = jnp.zeros_like(out_ref)

    out_ref[...] += jnp.sum(h_ref[...], axis=1)


def _tc_pool_body2(h_ref, out_ref):
    @pl.when(pl.program_id(1) == 0)
    def _():
        out_ref[...] = jnp.zeros_like(out_ref)

    out_ref[...] += jnp.sum(h_ref[...], axis=1)[:, None, :]


def _tc_pool(h3):
    out = pl.pallas_call(
        _tc_pool_body2,
        grid=(B, NT),
        in_specs=[
            pl.BlockSpec((1, CS, H), lambda b, i: (b, S_SC // CS + i, 0)),
        ],
        out_specs=pl.BlockSpec((1, 1, H), lambda b, i: (b, 0, 0)),
        out_shape=jax.ShapeDtypeStruct((B, 1, H), jnp.float32),
    )(h3)
    return out.reshape(B, H)


def _gate_body(p_ref, t_ref, w_ref, b_ref, loss_ref, idx_ref):
    pooled = (jnp.sum(p_ref[...], axis=1) + t_ref[...]) * (1.0 / S)
    logits = (
        jnp.dot(pooled, w_ref[...], preferred_element_type=jnp.float32)
        + b_ref[...]
    )
    m = jnp.max(logits, axis=1, keepdims=True)
    e = jnp.exp(logits - m)
    probs = e / jnp.sum(e, axis=1, keepdims=True)
    entropy = -jnp.sum(probs * jnp.log(probs + 1e-10)) * (1.0 / B)
    loss_ref[...] = jnp.reshape(-0.01 * entropy, (1, 1))
    row0 = probs[0:1, :]
    iota = lax.broadcasted_iota(jnp.int32, (1, D), 1)
    mx = jnp.max(row0)
    idx_ref[...] = jnp.reshape(jnp.min(jnp.where(row0 == mx, iota, D)), (1, 1))


def kernel(hidden_states, gates_W, gates_b, current_depth):
    h2 = hidden_states.reshape(B * S, H)
    sc_partials = _make_pool_sc()(h2)
    tc_sums = _tc_pool(hidden_states)
    w_d = lax.dynamic_index_in_dim(gates_W, current_depth, 0, keepdims=False)
    b_d = lax.dynamic_index_in_dim(gates_b, current_depth, 0, keepdims=True)
    loss, idx = pl.pallas_call(
        _gate_body,
        out_shape=(
            jax.ShapeDtypeStruct((1, 1), jnp.float32),
            jax.ShapeDtypeStruct((1, 1), jnp.int32),
        ),
    )(sc_partials.reshape(B, WPB, H), tc_sums, w_d, b_d)
    return (loss[0, 0], idx[0, 0])


# speedup vs baseline: 1.6930x; 1.0115x over previous
import functools

import jax
import jax.numpy as jnp
from jax import lax
from jax.experimental import pallas as pl
from jax.experimental.pallas import tpu as pltpu
from jax.experimental.pallas import tpu_sc as plsc

B, S, H, D = 4, 8192, 2048, 16
NC, NS, L = 2, 16, 16
NW = NC * NS
WPB = NW // B

S_SC = 1024
SPW = S_SC // WPB
RCH = 16
NCH = SPW // RCH

CS = 1024
NT = (S - S_SC) // CS


@functools.cache
def _make_pool_sc():
    mesh = plsc.VectorSubcoreMesh(core_axis_name="c", subcore_axis_name="s")
    return functools.partial(
        pl.kernel,
        mesh=mesh,
        out_type=jax.ShapeDtypeStruct((NW, H), jnp.float32),
        scratch_types=[
            pltpu.VMEM((RCH, H), jnp.float32),
            pltpu.VMEM((RCH, H), jnp.float32),
            pltpu.VMEM((H,), jnp.float32),
            pltpu.SemaphoreType.DMA,
            pltpu.SemaphoreType.DMA,
        ],
        compiler_params=pltpu.CompilerParams(vmem_limit_bytes=2 * 1024 * 1024),
    )(_pool_sc_body)


def _pool_sc_body(h_hbm, out_hbm, buf0, buf1, acc, sem0, sem1):
    cid = lax.axis_index("c")
    sid = lax.axis_index("s")
    wid = sid * NC + cid
    b = wid // WPB
    base = b * S + (wid % WPB) * SPW

    def _start(buf, sem, chunk):
        pltpu.make_async_copy(
            h_hbm.at[pl.ds(base + chunk * RCH, RCH)], buf, sem
        ).start()

    def _wait(buf, sem):
        pltpu.make_async_copy(h_hbm.at[pl.ds(base, RCH)], buf, sem).wait()

    def _zero(g, _):
        acc[pl.ds(g * L, L)] = jnp.zeros((L,), jnp.float32)
        return 0

    lax.fori_loop(0, H // L, _zero, 0)

    def _accum(buf):
        GU = 2

        def _one(g):
            col = pl.ds(g * L, L)
            x = [buf[r, col] for r in range(RCH)]
            while len(x) > 1:
                x = [x[i] + x[i + 1] for i in range(0, len(x), 2)]
            acc[col] = acc[col] + x[0]

        def _g(g, _):
            for u in range(GU):
                _one(g * GU + u)
            return 0

        lax.fori_loop(0, H // (L * GU), _g, 0)

    _start(buf0, sem0, 0)

    def _body(i, _):
        c0 = i * 2
        _start(buf1, sem1, c0 + 1)
        _wait(buf0, sem0)
        _accum(buf0)

        @pl.when(i < NCH // 2 - 1)
        def _():
            _start(buf0, sem0, c0 + 2)

        _wait(buf1, sem1)
        _accum(buf1)
        return 0

    lax.fori_loop(0, NCH // 2, _body, 0)
    pltpu.sync_copy(acc, out_hbm.at[wid])


def _tc_pool_body(h_ref, out_ref):
    @pl.when(pl.program_id(0) == 0)
    def _():
        out_ref[...] = jnp.zeros_like(out_ref)

    out_ref[...] += jnp.sum(h_ref[...], axis=1)


def _tc_pool_body2(h_ref, out_ref):
    @pl.when(pl.program_id(1) == 0)
    def _():
        out_ref[...] = jnp.zeros_like(out_ref)

    out_ref[...] += jnp.sum(h_ref[...], axis=1)[:, None, :]


def _tc_pool(h3):
    out = pl.pallas_call(
        _tc_pool_body2,
        grid=(B, NT),
        compiler_params=pltpu.CompilerParams(vmem_limit_bytes=24 * 1024 * 1024),
        in_specs=[
            pl.BlockSpec((1, CS, H), lambda b, i: (b, S_SC // CS + i, 0)),
        ],
        out_specs=pl.BlockSpec((1, 1, H), lambda b, i: (b, 0, 0)),
        out_shape=jax.ShapeDtypeStruct((B, 1, H), jnp.float32),
    )(h3)
    return out.reshape(B, H)


def _gate_body(p_ref, t_ref, w_ref, b_ref, loss_ref, idx_ref):
    pooled = (jnp.sum(p_ref[...], axis=1) + t_ref[...]) * (1.0 / S)
    logits = (
        jnp.dot(pooled, w_ref[...], preferred_element_type=jnp.float32)
        + b_ref[...]
    )
    m = jnp.max(logits, axis=1, keepdims=True)
    e = jnp.exp(logits - m)
    probs = e / jnp.sum(e, axis=1, keepdims=True)
    entropy = -jnp.sum(probs * jnp.log(probs + 1e-10)) * (1.0 / B)
    loss_ref[...] = jnp.reshape(-0.01 * entropy, (1, 1))
    row0 = probs[0:1, :]
    iota = lax.broadcasted_iota(jnp.int32, (1, D), 1)
    mx = jnp.max(row0)
    idx_ref[...] = jnp.reshape(jnp.min(jnp.where(row0 == mx, iota, D)), (1, 1))


def kernel(hidden_states, gates_W, gates_b, current_depth):
    h2 = hidden_states.reshape(B * S, H)
    sc_partials = _make_pool_sc()(h2)
    tc_sums = _tc_pool(hidden_states)
    w_d = lax.dynamic_index_in_dim(gates_W, current_depth, 0, keepdims=False)
    b_d = lax.dynamic_index_in_dim(gates_b, current_depth, 0, keepdims=True)
    loss, idx = pl.pallas_call(
        _gate_body,
        out_shape=(
            jax.ShapeDtypeStruct((1, 1), jnp.float32),
            jax.ShapeDtypeStruct((1, 1), jnp.int32),
        ),
    )(sc_partials.reshape(B, WPB, H), tc_sums, w_d, b_d)
    return (loss[0, 0], idx[0, 0])


# speedup vs baseline: 1.6958x; 1.0017x over previous
import functools

import jax
import jax.numpy as jnp
from jax import lax
from jax.experimental import pallas as pl
from jax.experimental.pallas import tpu as pltpu
from jax.experimental.pallas import tpu_sc as plsc

B, S, H, D = 4, 8192, 2048, 16
NC, NS, L = 2, 16, 16
NW = NC * NS
WPB = NW // B

S_SC = 512
SPW = S_SC // WPB
RCH = 16
NCH = SPW // RCH

CS = 2560
NT = (S - S_SC) // CS


@functools.cache
def _make_pool_sc():
    mesh = plsc.VectorSubcoreMesh(core_axis_name="c", subcore_axis_name="s")
    return functools.partial(
        pl.kernel,
        mesh=mesh,
        out_type=jax.ShapeDtypeStruct((NW, H), jnp.float32),
        scratch_types=[
            pltpu.VMEM((RCH, H), jnp.float32),
            pltpu.VMEM((RCH, H), jnp.float32),
            pltpu.VMEM((H,), jnp.float32),
            pltpu.SemaphoreType.DMA,
            pltpu.SemaphoreType.DMA,
        ],
        compiler_params=pltpu.CompilerParams(vmem_limit_bytes=2 * 1024 * 1024),
    )(_pool_sc_body)


def _pool_sc_body(h_hbm, out_hbm, buf0, buf1, acc, sem0, sem1):
    cid = lax.axis_index("c")
    sid = lax.axis_index("s")
    wid = sid * NC + cid
    b = wid // WPB
    base = b * S + (S - S_SC) + (wid % WPB) * SPW

    def _start(buf, sem, chunk):
        pltpu.make_async_copy(
            h_hbm.at[pl.ds(base + chunk * RCH, RCH)], buf, sem
        ).start()

    def _wait(buf, sem):
        pltpu.make_async_copy(h_hbm.at[pl.ds(base, RCH)], buf, sem).wait()

    def _zero(g, _):
        acc[pl.ds(g * L, L)] = jnp.zeros((L,), jnp.float32)
        return 0

    lax.fori_loop(0, H // L, _zero, 0)

    def _accum(buf):
        GU = 2

        def _one(g):
            col = pl.ds(g * L, L)
            x = [buf[r, col] for r in range(RCH)]
            while len(x) > 1:
                x = [x[i] + x[i + 1] for i in range(0, len(x), 2)]
            acc[col] = acc[col] + x[0]

        def _g(g, _):
            for u in range(GU):
                _one(g * GU + u)
            return 0

        lax.fori_loop(0, H // (L * GU), _g, 0)

    _start(buf0, sem0, 0)

    def _body(i, _):
        c0 = i * 2
        _start(buf1, sem1, c0 + 1)
        _wait(buf0, sem0)
        _accum(buf0)

        @pl.when(i < NCH // 2 - 1)
        def _():
            _start(buf0, sem0, c0 + 2)

        _wait(buf1, sem1)
        _accum(buf1)
        return 0

    lax.fori_loop(0, NCH // 2, _body, 0)
    pltpu.sync_copy(acc, out_hbm.at[wid])


def _tc_pool_body(h_ref, out_ref):
    @pl.when(pl.program_id(0) == 0)
    def _():
        out_ref[...] = jnp.zeros_like(out_ref)

    out_ref[...] += jnp.sum(h_ref[...], axis=1)


def _tc_pool_body2(h_ref, out_ref):
    @pl.when(pl.program_id(1) == 0)
    def _():
        out_ref[...] = jnp.zeros_like(out_ref)

    out_ref[...] += jnp.sum(h_ref[...], axis=1)[:, None, :]


def _tc_pool(h3):
    out = pl.pallas_call(
        _tc_pool_body2,
        grid=(B, NT),
        compiler_params=pltpu.CompilerParams(vmem_limit_bytes=46 * 1024 * 1024),
        in_specs=[
            pl.BlockSpec((1, CS, H), lambda b, i: (b, i, 0)),
        ],
        out_specs=pl.BlockSpec((1, 1, H), lambda b, i: (b, 0, 0)),
        out_shape=jax.ShapeDtypeStruct((B, 1, H), jnp.float32),
    )(h3)
    return out.reshape(B, H)


def _gate_body(p_ref, t_ref, w_ref, b_ref, loss_ref, idx_ref):
    pooled = (jnp.sum(p_ref[...], axis=1) + t_ref[...]) * (1.0 / S)
    logits = (
        jnp.dot(pooled, w_ref[...], preferred_element_type=jnp.float32)
        + b_ref[...]
    )
    m = jnp.max(logits, axis=1, keepdims=True)
    e = jnp.exp(logits - m)
    probs = e / jnp.sum(e, axis=1, keepdims=True)
    entropy = -jnp.sum(probs * jnp.log(probs + 1e-10)) * (1.0 / B)
    loss_ref[...] = jnp.reshape(-0.01 * entropy, (1, 1))
    row0 = probs[0:1, :]
    iota = lax.broadcasted_iota(jnp.int32, (1, D), 1)
    mx = jnp.max(row0)
    idx_ref[...] = jnp.reshape(jnp.min(jnp.where(row0 == mx, iota, D)), (1, 1))


def kernel(hidden_states, gates_W, gates_b, current_depth):
    h2 = hidden_states.reshape(B * S, H)
    sc_partials = _make_pool_sc()(h2)
    tc_sums = _tc_pool(hidden_states)
    w_d = lax.dynamic_index_in_dim(gates_W, current_depth, 0, keepdims=False)
    b_d = lax.dynamic_index_in_dim(gates_b, current_depth, 0, keepdims=True)
    loss, idx = pl.pallas_call(
        _gate_body,
        out_shape=(
            jax.ShapeDtypeStruct((1, 1), jnp.float32),
            jax.ShapeDtypeStruct((1, 1), jnp.int32),
        ),
    )(sc_partials.reshape(B, WPB, H), tc_sums, w_d, b_d)
    return (loss[0, 0], idx[0, 0])
